# baseline (speedup 1.0000x reference)
import dataclasses
import functools

import jax
import jax.numpy as jnp
from jax import lax
from jax.experimental import pallas as pl
from jax.experimental.pallas import tpu as pltpu
from jax.experimental.pallas import tpu_sc as plsc

Q = 4096
D = 768
K = 50000
KT = 512
GRID_K = 98
KPAD = KT * GRID_K
SEG = 128
G = KPAD // SEG
SEG_PER_STEP = KT // SEG
KNN = 9
NSEL = 16
PAD_SEG = G - 1
BIG = 1e30

NW = 32
QPW = Q // NW
QB = 8
NB = QPW // QB
LANES = 16


def _cdist_body(a_ref, b_ref, d2_ref, m_ref, abf_ref):
    k = pl.program_id(0)

    @pl.when(k == 0)
    def _init():
        abf_ref[...] = a_ref[...].astype(jnp.bfloat16)

    bbf = b_ref[...].astype(jnp.bfloat16)
    bsq = bbf * bbf
    ones_row = jnp.ones((8, D), jnp.bfloat16)
    m2 = lax.dot_general(
        ones_row, bsq, (((1,), (1,)), ((), ())),
        preferred_element_type=jnp.float32)[:1]
    ab = lax.dot_general(
        abf_ref[...], bbf,
        (((1,), (1,)), ((), ())),
        preferred_element_type=jnp.float32)
    e2 = m2 - 2.0 * ab

    def _mask(x):
        col = k * KT + lax.broadcasted_iota(jnp.int32, (Q, KT), 1)
        return jnp.where(col < K, x, BIG)

    e2 = lax.cond(k == GRID_K - 1, _mask, lambda x: x, e2)
    m_ref[...] = jnp.min(e2.reshape(Q, SEG_PER_STEP, SEG), axis=2)[None]
    e2r = e2.reshape(Q // 8, 8, KT)
    for j in range(SEG_PER_STEP):
        d2_ref[:, j] = e2r[:, :, j * SEG:(j + 1) * SEG]


def _cdist_call(features, memory_bank):
    return pl.pallas_call(
        _cdist_body,
        grid=(GRID_K,),
        in_specs=[
            pl.BlockSpec((Q, D), lambda k: (0, 0)),
            pl.BlockSpec((KT, D), lambda k: (k, 0)),
        ],
        out_specs=[
            pl.BlockSpec((Q // 8, SEG_PER_STEP, 8, SEG), lambda k: (0, k, 0, 0)),
            pl.BlockSpec((1, Q, SEG_PER_STEP), lambda k: (k, 0, 0)),
        ],
        out_shape=[
            jax.ShapeDtypeStruct((Q // 8, G, 8, SEG), jnp.float32),
            jax.ShapeDtypeStruct((GRID_K, Q, SEG_PER_STEP), jnp.float32),
        ],
        scratch_shapes=[
            pltpu.VMEM((Q, D), jnp.bfloat16),
        ],
        compiler_params=pltpu.CompilerParams(
            dimension_semantics=("arbitrary",)),
    )(features, memory_bank)


def _select_body(m_ref, idx_ref):
    m = m_ref[...]
    colg = lax.broadcasted_iota(jnp.int32, (Q, G), 1)
    lane = lax.broadcasted_iota(jnp.int32, (Q, NSEL), 1)
    qrow = lax.broadcasted_iota(jnp.int32, (Q, NSEL), 0)
    out = jnp.full((Q, NSEL), PAD_SEG, jnp.int32)
    cur = m
    for j in range(KNN):
        mn = jnp.min(cur, axis=1, keepdims=True)
        am = jnp.min(jnp.where(cur == mn, colg, G), axis=1, keepdims=True)
        out = jnp.where(lane == j, am, out)
        cur = jnp.where(colg == am, BIG, cur)
    idx_ref[...] = ((qrow // 8) * G + out) * 8 + (qrow % 8)


def _select_call(m):
    return pl.pallas_call(
        _select_body,
        in_specs=[pl.BlockSpec((Q, G), lambda: (0, 0))],
        out_specs=pl.BlockSpec((Q, NSEL), lambda: (0, 0)),
        out_shape=jax.ShapeDtypeStruct((Q, NSEL), jnp.int32),
    )(m)


def _sc_topk(d2_rows, idx_flat):
    mesh = plsc.VectorSubcoreMesh(core_axis_name="c", subcore_axis_name="s")
    cp = pltpu.CompilerParams()
    if "needs_layout_passes" in pltpu.CompilerParams.__dataclass_fields__:
        cp = dataclasses.replace(cp, needs_layout_passes=False)

    @functools.partial(
        pl.kernel,
        mesh=mesh,
        compiler_params=cp,
        out_type=jax.ShapeDtypeStruct((Q * NSEL,), jnp.float32),
        scratch_types=[
            pltpu.VMEM((QB * NSEL,), jnp.int32),
            pltpu.VMEM((QB * NSEL, SEG), jnp.float32),
            pltpu.VMEM((QB * NSEL,), jnp.float32),
            pltpu.SemaphoreType.DMA,
        ],
    )
    def body(d2_hbm, idx_hbm, out_hbm, idx_v, rows_v, out_v, sem):
        wid = lax.axis_index("s") * 2 + lax.axis_index("c")
        base = wid * (QPW * NSEL)

        @pl.loop(0, NB)
        def _batch(bi):
            off = base + bi * (QB * NSEL)
            pltpu.sync_copy(idx_hbm.at[pl.ds(off, QB * NSEL)], idx_v)
            pltpu.async_copy(d2_hbm.at[idx_v], rows_v, sem).wait()

            @pl.loop(0, QB)
            def _query(q):
                acc = jnp.full((LANES,), BIG, jnp.float32)
                for r in range(KNN):
                    for c in range(SEG // LANES):
                        v = rows_v[q * NSEL + r, pl.ds(c * LANES, LANES)]
                        s = lax.sort(v)
                        acc = lax.sort(jnp.minimum(acc, lax.rev(s, (0,))))
                out_v[pl.ds(q * NSEL, LANES)] = acc

            pltpu.sync_copy(out_v, out_hbm.at[pl.ds(off, QB * NSEL)])

    return body(d2_rows, idx_flat)


def _q2_body(a_ref, o_ref):
    a = a_ref[...]
    o_ref[...] = jnp.sum(a * a, axis=1, keepdims=True)


def _q2_call(features):
    return pl.pallas_call(
        _q2_body,
        in_specs=[pl.BlockSpec((Q, D), lambda: (0, 0))],
        out_specs=pl.BlockSpec((Q, 1), lambda: (0, 0)),
        out_shape=jax.ShapeDtypeStruct((Q, 1), jnp.float32),
    )(features)


def _mean_body(t_ref, q2_ref, o_ref):
    t = t_ref[...] + q2_ref[...]
    lane = lax.broadcasted_iota(jnp.int32, (Q, NSEL), 1)
    dist = jnp.sqrt(jnp.maximum(t, 1e-12))
    o_ref[...] = jnp.sum(jnp.where(lane < KNN, dist, 0.0), axis=1,
                         keepdims=True) * (1.0 / KNN)


def _mean_call(t, q2):
    return pl.pallas_call(
        _mean_body,
        in_specs=[pl.BlockSpec((Q, NSEL), lambda: (0, 0)),
                  pl.BlockSpec((Q, 1), lambda: (0, 0))],
        out_specs=pl.BlockSpec((Q, 1), lambda: (0, 0)),
        out_shape=jax.ShapeDtypeStruct((Q, 1), jnp.float32),
    )(t, q2)


def kernel(features, memory_bank):
    d2, m3 = _cdist_call(features, memory_bank)
    q2 = _q2_call(features)
    m = jnp.transpose(m3, (1, 0, 2)).reshape(Q, G)
    idx = _select_call(m)
    t9 = _sc_topk(d2.reshape((Q // 8) * G * 8, SEG), idx.reshape(Q * NSEL))
    out = _mean_call(t9.reshape(Q, NSEL), q2)
    return out.reshape(Q)


# speedup vs baseline: 1.3362x; 1.3362x over previous
import dataclasses
import functools

import jax
import jax.numpy as jnp
from jax import lax
from jax.experimental import pallas as pl
from jax.experimental.pallas import tpu as pltpu
from jax.experimental.pallas import tpu_sc as plsc

Q = 4096
D = 768
K = 50000
KT = 512
GRID_K = 98
KPAD = KT * GRID_K
SEG = 128
G = KPAD // SEG
SEG_PER_STEP = KT // SEG
KNN = 9
NSEL = 16
PAD_SEG = G - 1
BIG = 1e30

NW = 32
QPW = Q // NW
QB = 8
NB = QPW // QB
LANES = 16


def _cdist_body(a_ref, b_ref, d2_ref, m_ref, abf_ref):
    k = pl.program_id(0)

    @pl.when(k == 0)
    def _init():
        abf_ref[...] = a_ref[...].astype(jnp.bfloat16)

    bbf = b_ref[...].astype(jnp.bfloat16)
    bsq = bbf * bbf
    ones_row = jnp.ones((8, D), jnp.bfloat16)
    m2 = lax.dot_general(
        ones_row, bsq, (((1,), (1,)), ((), ())),
        preferred_element_type=jnp.float32)[:1]
    ab = lax.dot_general(
        abf_ref[...], bbf,
        (((1,), (1,)), ((), ())),
        preferred_element_type=jnp.float32)
    e2 = m2 - 2.0 * ab
    lane_col = lax.broadcasted_iota(jnp.int32, (Q, KT), 1)
    e2 = jnp.where(lane_col < K - k * KT, e2, BIG)
    m_ref[...] = jnp.min(e2.reshape(Q, SEG_PER_STEP, SEG), axis=2)[None]
    e2r = e2.reshape(Q // 8, 8, KT)
    for j in range(SEG_PER_STEP):
        d2_ref[:, j] = e2r[:, :, j * SEG:(j + 1) * SEG]


def _cdist_call(features, memory_bank):
    return pl.pallas_call(
        _cdist_body,
        grid=(GRID_K,),
        in_specs=[
            pl.BlockSpec((Q, D), lambda k: (0, 0)),
            pl.BlockSpec((KT, D), lambda k: (k, 0)),
        ],
        out_specs=[
            pl.BlockSpec((Q // 8, SEG_PER_STEP, 8, SEG), lambda k: (0, k, 0, 0)),
            pl.BlockSpec((1, Q, SEG_PER_STEP), lambda k: (k, 0, 0)),
        ],
        out_shape=[
            jax.ShapeDtypeStruct((Q // 8, G, 8, SEG), jnp.float32),
            jax.ShapeDtypeStruct((GRID_K, Q, SEG_PER_STEP), jnp.float32),
        ],
        scratch_shapes=[
            pltpu.VMEM((Q, D), jnp.bfloat16),
        ],
        compiler_params=pltpu.CompilerParams(
            dimension_semantics=("arbitrary",)),
    )(features, memory_bank)


def _select_body(m_ref, idx_ref):
    m = m_ref[...]
    colg = lax.broadcasted_iota(jnp.int32, (Q, G), 1)
    lane = lax.broadcasted_iota(jnp.int32, (Q, NSEL), 1)
    qrow = lax.broadcasted_iota(jnp.int32, (Q, NSEL), 0)
    out = jnp.full((Q, NSEL), PAD_SEG, jnp.int32)
    cur = m
    for j in range(KNN):
        mn = jnp.min(cur, axis=1, keepdims=True)
        am = jnp.min(jnp.where(cur == mn, colg, G), axis=1, keepdims=True)
        out = jnp.where(lane == j, am, out)
        cur = jnp.where(colg == am, BIG, cur)
    idx_ref[...] = ((qrow // 8) * G + out) * 8 + (qrow % 8)


def _select_call(m):
    return pl.pallas_call(
        _select_body,
        in_specs=[pl.BlockSpec((Q, G), lambda: (0, 0))],
        out_specs=pl.BlockSpec((Q, NSEL), lambda: (0, 0)),
        out_shape=jax.ShapeDtypeStruct((Q, NSEL), jnp.int32),
    )(m)


def _sc_topk(d2_rows, idx_flat):
    mesh = plsc.VectorSubcoreMesh(core_axis_name="c", subcore_axis_name="s")
    cp = pltpu.CompilerParams()
    if "needs_layout_passes" in pltpu.CompilerParams.__dataclass_fields__:
        cp = dataclasses.replace(cp, needs_layout_passes=False)

    @functools.partial(
        pl.kernel,
        mesh=mesh,
        compiler_params=cp,
        out_type=jax.ShapeDtypeStruct((Q * NSEL,), jnp.float32),
        scratch_types=[
            pltpu.VMEM((QB * NSEL,), jnp.int32),
            pltpu.VMEM((QB * NSEL, SEG), jnp.float32),
            pltpu.VMEM((QB * NSEL,), jnp.float32),
            pltpu.SemaphoreType.DMA,
        ],
    )
    def body(d2_hbm, idx_hbm, out_hbm, idx_v, rows_v, out_v, sem):
        wid = lax.axis_index("s") * 2 + lax.axis_index("c")
        base = wid * (QPW * NSEL)

        @pl.loop(0, NB)
        def _batch(bi):
            off = base + bi * (QB * NSEL)
            pltpu.sync_copy(idx_hbm.at[pl.ds(off, QB * NSEL)], idx_v)
            pltpu.async_copy(d2_hbm.at[idx_v], rows_v, sem).wait()

            @pl.loop(0, QB)
            def _query(q):
                acc = jnp.full((LANES,), BIG, jnp.float32)
                for r in range(KNN):
                    for c in range(SEG // LANES):
                        v = rows_v[q * NSEL + r, pl.ds(c * LANES, LANES)]
                        s = lax.sort(v)
                        acc = lax.sort(jnp.minimum(acc, lax.rev(s, (0,))))
                out_v[pl.ds(q * NSEL, LANES)] = acc

            pltpu.sync_copy(out_v, out_hbm.at[pl.ds(off, QB * NSEL)])

    return body(d2_rows, idx_flat)


def _q2_body(a_ref, o_ref):
    a = a_ref[...]
    o_ref[...] = jnp.sum(a * a, axis=1, keepdims=True)


def _q2_call(features):
    return pl.pallas_call(
        _q2_body,
        in_specs=[pl.BlockSpec((Q, D), lambda: (0, 0))],
        out_specs=pl.BlockSpec((Q, 1), lambda: (0, 0)),
        out_shape=jax.ShapeDtypeStruct((Q, 1), jnp.float32),
    )(features)


def _mean_body(t_ref, q2_ref, o_ref):
    t = t_ref[...] + q2_ref[...]
    lane = lax.broadcasted_iota(jnp.int32, (Q, NSEL), 1)
    dist = jnp.sqrt(jnp.maximum(t, 1e-12))
    o_ref[...] = jnp.sum(jnp.where(lane < KNN, dist, 0.0), axis=1,
                         keepdims=True) * (1.0 / KNN)


def _mean_call(t, q2):
    return pl.pallas_call(
        _mean_body,
        in_specs=[pl.BlockSpec((Q, NSEL), lambda: (0, 0)),
                  pl.BlockSpec((Q, 1), lambda: (0, 0))],
        out_specs=pl.BlockSpec((Q, 1), lambda: (0, 0)),
        out_shape=jax.ShapeDtypeStruct((Q, 1), jnp.float32),
    )(t, q2)


def kernel(features, memory_bank):
    d2, m3 = _cdist_call(features, memory_bank)
    q2 = _q2_call(features)
    m = jnp.transpose(m3, (1, 0, 2)).reshape(Q, G)
    idx = _select_call(m)
    t9 = _sc_topk(d2.reshape((Q // 8) * G * 8, SEG), idx.reshape(Q * NSEL))
    out = _mean_call(t9.reshape(Q, NSEL), q2)
    return out.reshape(Q)
